# baseline (speedup 1.0000x reference)
import functools

import jax
import jax.numpy as jnp
from jax import lax
from jax.experimental import pallas as pl
from jax.experimental.pallas import tpu as pltpu
from jax.experimental.pallas import tpu_sc as plsc

_R = 256


def _cumsum_sub(a, n):
    k = 1
    while k < n:
        pad = jnp.zeros((k,) + a.shape[1:], a.dtype)
        a = a + jnp.concatenate([pad, a[: n - k]], axis=0)
        k *= 2
    return a


def _cumsum_lane(a, n):
    k = 1
    while k < n:
        pad = jnp.zeros(a.shape[:1] + (k,), a.dtype)
        a = a + jnp.concatenate([pad, a[:, : n - k]], axis=1)
        k *= 2
    return a


def _gate_route_kernel(x_ref, gw_ref, d0_ref, d1_ref, w0_ref, w1_ref,
                       te_ref, act_ref, *, r_tile, n_tiles):
    t = x_ref.shape[0]
    n_e = gw_ref.shape[0]
    logits = jax.lax.dot_general(
        x_ref[...].astype(jnp.bfloat16), gw_ref[...].astype(jnp.bfloat16),
        (((1,), (1,)), ((), ())),
        preferred_element_type=jnp.float32)
    m = jnp.max(logits, axis=1, keepdims=True)
    ex = jnp.exp(logits - m)
    scores = ex / jnp.sum(ex, axis=1, keepdims=True)

    idx = jax.lax.broadcasted_iota(jnp.int32, scores.shape, 1)
    big = jnp.int32(n_e + 1)
    m1 = jnp.max(scores, axis=1, keepdims=True)
    i1 = jnp.min(jnp.where(scores == m1, idx, big), axis=1, keepdims=True)
    scores2 = jnp.where(idx == i1, jnp.float32(-1.0), scores)
    m2 = jnp.max(scores2, axis=1, keepdims=True)
    i2 = jnp.min(jnp.where(scores2 == m2, idx, big), axis=1, keepdims=True)
    w0_ref[...] = m1
    w1_ref[...] = m2

    sel = jnp.logical_or(idx == i1, idx == i2).astype(jnp.int32)
    c_inc = _cumsum_sub(sel, t)
    c_exc = c_inc - sel
    counts = c_inc[t - 1 : t, :]
    pc = ((counts + (r_tile - 1)) // r_tile) * r_tile
    base = _cumsum_lane(pc, n_e) - pc
    slot = c_exc + base
    d0_ref[...] = jnp.sum(jnp.where(idx == i1, slot, 0), axis=1,
                          keepdims=True)
    d1_ref[...] = jnp.sum(jnp.where(idx == i2, slot, 0), axis=1,
                          keepdims=True)

    tb = base // r_tile
    ntl = pc // r_tile
    tio = jax.lax.broadcasted_iota(jnp.int32, (n_tiles, n_e), 0)
    in_e = jnp.logical_and(tio >= tb, tio < tb + ntl).astype(jnp.int32)
    eio = jax.lax.broadcasted_iota(jnp.int32, (n_tiles, n_e), 1)
    te_ref[...] = jnp.sum(in_e * eio, axis=1, keepdims=True)
    act_ref[...] = jnp.sum(in_e, axis=1, keepdims=True)


def _grouped_kernel(te_ref, act_ref, xs_ref, w1_ref, w3_ref, w2_ref,
                    ys_ref, acc_ref):
    i = pl.program_id(0)
    s = pl.program_id(1)

    @pl.when(act_ref[i] == 1)
    def _():
        xb = xs_ref[...].astype(jnp.bfloat16)
        a1 = jax.lax.dot_general(
            xb, w1_ref[0], (((1,), (1,)), ((), ())),
            preferred_element_type=jnp.float32)
        a3 = jax.lax.dot_general(
            xb, w3_ref[0], (((1,), (1,)), ((), ())),
            preferred_element_type=jnp.float32)
        ch = ((a1 * jax.nn.sigmoid(a1)) * a3).astype(jnp.bfloat16)

        @pl.when(s == 0)
        def _():
            acc_ref[...] = ch

        @pl.when(s == 1)
        def _():
            av = jnp.concatenate([acc_ref[...], ch], axis=1)
            ys_ref[...] = jax.lax.dot_general(
                av, w2_ref[0], (((1,), (1,)), ((), ())),
                preferred_element_type=jnp.float32)


def _shared_kernel(x_ref, w1_ref, w3_ref, w2_ref, out_ref, acc_ref):
    s = pl.program_id(1)

    xb = x_ref[...].astype(jnp.bfloat16)
    a1 = jax.lax.dot_general(
        xb, w1_ref[...], (((1,), (1,)), ((), ())),
        preferred_element_type=jnp.float32)
    a3 = jax.lax.dot_general(
        xb, w3_ref[...], (((1,), (1,)), ((), ())),
        preferred_element_type=jnp.float32)
    ch = ((a1 * jax.nn.sigmoid(a1)) * a3).astype(jnp.bfloat16)

    @pl.when(s == 0)
    def _():
        acc_ref[...] = ch

    @pl.when(s == 1)
    def _():
        av = jnp.concatenate([acc_ref[...], ch], axis=1)
        out_ref[...] = jax.lax.dot_general(
            av, w2_ref[...], (((1,), (1,)), ((), ())),
            preferred_element_type=jnp.float32)


def _combine_kernel(sh_ref, g0_ref, g1_ref, w0_ref, w1_ref, out_ref):
    out_ref[...] = (sh_ref[...] + w0_ref[...] * g0_ref[...]
                    + w1_ref[...] * g1_ref[...])


def _sc_dispatch(x, d0f, d1f, npad):
    t, d = x.shape
    nc, ns = 2, 16
    nw = nc * ns
    ch = t // nw
    mesh = plsc.VectorSubcoreMesh(core_axis_name="c", subcore_axis_name="s")

    @functools.partial(
        pl.kernel, mesh=mesh,
        out_type=jax.ShapeDtypeStruct((npad, d), jnp.float32),
        scratch_types=[pltpu.VMEM((ch, d), jnp.float32),
                       pltpu.VMEM((ch,), jnp.int32),
                       pltpu.SemaphoreType.DMA])
    def k(x_hbm, d0_hbm, d1_hbm, xs_hbm, xbuf, ibuf, sem):
        wid = lax.axis_index("s") * nc + lax.axis_index("c")
        base = wid * ch
        pltpu.async_copy(x_hbm.at[pl.ds(base, ch)], xbuf, sem).wait()
        pltpu.async_copy(d0_hbm.at[pl.ds(base, ch)], ibuf, sem).wait()
        pltpu.async_copy(xbuf, xs_hbm.at[ibuf], sem).wait()
        pltpu.async_copy(d1_hbm.at[pl.ds(base, ch)], ibuf, sem).wait()
        pltpu.async_copy(xbuf, xs_hbm.at[ibuf], sem).wait()

    return k(x, d0f, d1f)


def _sc_combine(ys, d0f, d1f):
    t = d0f.shape[0]
    d = ys.shape[1]
    nc, ns = 2, 16
    nw = nc * ns
    ch = t // nw
    mesh = plsc.VectorSubcoreMesh(core_axis_name="c", subcore_axis_name="s")

    @functools.partial(
        pl.kernel, mesh=mesh,
        out_type=[jax.ShapeDtypeStruct((t, d), jnp.float32),
                  jax.ShapeDtypeStruct((t, d), jnp.float32)],
        scratch_types=[pltpu.VMEM((ch, d), jnp.float32),
                       pltpu.VMEM((ch,), jnp.int32),
                       pltpu.SemaphoreType.DMA])
    def k(ys_hbm, d0_hbm, d1_hbm, g0_hbm, g1_hbm, buf, ibuf, sem):
        wid = lax.axis_index("s") * nc + lax.axis_index("c")
        base = wid * ch
        pltpu.async_copy(d0_hbm.at[pl.ds(base, ch)], ibuf, sem).wait()
        pltpu.async_copy(ys_hbm.at[ibuf], buf, sem).wait()
        pltpu.async_copy(buf, g0_hbm.at[pl.ds(base, ch)], sem).wait()
        pltpu.async_copy(d1_hbm.at[pl.ds(base, ch)], ibuf, sem).wait()
        pltpu.async_copy(ys_hbm.at[ibuf], buf, sem).wait()
        pltpu.async_copy(buf, g1_hbm.at[pl.ds(base, ch)], sem).wait()

    return k(ys, d0f, d1f)


@jax.jit
def kernel(x, gate_w, w1, w3, w2, ws1, ws3, ws2):
    t, dim = x.shape
    n_exp, hid, _ = w1.shape
    n_k = 2
    n_tiles = (t * n_k) // _R + n_exp
    npad = n_tiles * _R

    d0, d1, w0c, w1c, te, act = pl.pallas_call(
        functools.partial(_gate_route_kernel, r_tile=_R, n_tiles=n_tiles),
        out_shape=[
            jax.ShapeDtypeStruct((t, 1), jnp.int32),
            jax.ShapeDtypeStruct((t, 1), jnp.int32),
            jax.ShapeDtypeStruct((t, 1), jnp.float32),
            jax.ShapeDtypeStruct((t, 1), jnp.float32),
            jax.ShapeDtypeStruct((n_tiles, 1), jnp.int32),
            jax.ShapeDtypeStruct((n_tiles, 1), jnp.int32),
        ],
    )(x, gate_w)

    d0f = d0.reshape(t)
    d1f = d1.reshape(t)
    te_f = te.reshape(n_tiles)
    act_f = act.reshape(n_tiles)

    xs = _sc_dispatch(x, d0f, d1f, npad)

    hid_p = hid

    tt = max(1, t // 256)
    t_blk = t // tt
    h_blk = hid_p // 2
    shared = pl.pallas_call(
        _shared_kernel,
        grid=(tt, 2),
        in_specs=[
            pl.BlockSpec((t_blk, dim), lambda i, h: (i, 0)),
            pl.BlockSpec((h_blk, dim), lambda i, h: (h, 0)),
            pl.BlockSpec((h_blk, dim), lambda i, h: (h, 0)),
            pl.BlockSpec((dim, hid_p), lambda i, h: (0, 0)),
        ],
        out_specs=pl.BlockSpec((t_blk, dim), lambda i, h: (i, 0)),
        out_shape=jax.ShapeDtypeStruct((t, dim), jnp.float32),
        scratch_shapes=[pltpu.VMEM((t_blk, h_blk), jnp.bfloat16)],
    )(x, ws1.astype(jnp.bfloat16), ws3.astype(jnp.bfloat16),
      ws2.astype(jnp.bfloat16))

    grid_spec = pltpu.PrefetchScalarGridSpec(
        num_scalar_prefetch=2,
        grid=(n_tiles, 2),
        in_specs=[
            pl.BlockSpec((_R, dim), lambda i, s, te_r, act_r: (i, 0)),
            pl.BlockSpec((1, h_blk, dim),
                         lambda i, s, te_r, act_r: (te_r[i], s, 0)),
            pl.BlockSpec((1, h_blk, dim),
                         lambda i, s, te_r, act_r: (te_r[i], s, 0)),
            pl.BlockSpec((1, dim, hid_p),
                         lambda i, s, te_r, act_r: (te_r[i], 0, 0)),
        ],
        out_specs=pl.BlockSpec((_R, dim), lambda i, s, te_r, act_r: (i, 0)),
        scratch_shapes=[pltpu.VMEM((_R, h_blk), jnp.bfloat16)],
    )
    ys = pl.pallas_call(
        _grouped_kernel,
        grid_spec=grid_spec,
        out_shape=jax.ShapeDtypeStruct((npad, dim), jnp.float32),
    )(te_f, act_f, xs, w1.astype(jnp.bfloat16), w3.astype(jnp.bfloat16),
      w2.astype(jnp.bfloat16))

    g0, g1 = _sc_combine(ys, d0f, d1f)

    out = pl.pallas_call(
        _combine_kernel,
        grid=(tt,),
        in_specs=[
            pl.BlockSpec((t_blk, dim), lambda i: (i, 0)),
            pl.BlockSpec((t_blk, dim), lambda i: (i, 0)),
            pl.BlockSpec((t_blk, dim), lambda i: (i, 0)),
            pl.BlockSpec((t_blk, 1), lambda i: (i, 0)),
            pl.BlockSpec((t_blk, 1), lambda i: (i, 0)),
        ],
        out_specs=pl.BlockSpec((t_blk, dim), lambda i: (i, 0)),
        out_shape=jax.ShapeDtypeStruct((t, dim), jnp.float32),
    )(shared, g0, g1, w0c, w1c)
    return out


# speedup vs baseline: 1.1273x; 1.1273x over previous
import functools

import jax
import jax.numpy as jnp
from jax import lax
from jax.experimental import pallas as pl
from jax.experimental.pallas import tpu as pltpu
from jax.experimental.pallas import tpu_sc as plsc

_R = 256


def _cumsum_sub(a, n):
    k = 1
    while k < n:
        pad = jnp.zeros((k,) + a.shape[1:], a.dtype)
        a = a + jnp.concatenate([pad, a[: n - k]], axis=0)
        k *= 2
    return a


def _cumsum_lane(a, n):
    k = 1
    while k < n:
        pad = jnp.zeros(a.shape[:1] + (k,), a.dtype)
        a = a + jnp.concatenate([pad, a[:, : n - k]], axis=1)
        k *= 2
    return a


def _gate_route_kernel(x_ref, gw_ref, d0_ref, d1_ref, w0_ref, w1_ref,
                       te_ref, act_ref, *, r_tile, n_tiles):
    t = x_ref.shape[0]
    n_e = gw_ref.shape[0]
    logits = jax.lax.dot_general(
        x_ref[...].astype(jnp.bfloat16), gw_ref[...].astype(jnp.bfloat16),
        (((1,), (1,)), ((), ())),
        preferred_element_type=jnp.float32)
    m = jnp.max(logits, axis=1, keepdims=True)
    ex = jnp.exp(logits - m)
    scores = ex / jnp.sum(ex, axis=1, keepdims=True)

    idx = jax.lax.broadcasted_iota(jnp.int32, scores.shape, 1)
    big = jnp.int32(n_e + 1)
    m1 = jnp.max(scores, axis=1, keepdims=True)
    i1 = jnp.min(jnp.where(scores == m1, idx, big), axis=1, keepdims=True)
    scores2 = jnp.where(idx == i1, jnp.float32(-1.0), scores)
    m2 = jnp.max(scores2, axis=1, keepdims=True)
    i2 = jnp.min(jnp.where(scores2 == m2, idx, big), axis=1, keepdims=True)
    w0_ref[...] = m1
    w1_ref[...] = m2

    sel = jnp.logical_or(idx == i1, idx == i2).astype(jnp.int32)
    c_inc = _cumsum_sub(sel, t)
    c_exc = c_inc - sel
    counts = c_inc[t - 1 : t, :]
    pc = ((counts + (r_tile - 1)) // r_tile) * r_tile
    base = _cumsum_lane(pc, n_e) - pc
    slot = c_exc + base
    d0_ref[...] = jnp.sum(jnp.where(idx == i1, slot, 0), axis=1,
                          keepdims=True)
    d1_ref[...] = jnp.sum(jnp.where(idx == i2, slot, 0), axis=1,
                          keepdims=True)

    tb = base // r_tile
    ntl = pc // r_tile
    tio = jax.lax.broadcasted_iota(jnp.int32, (n_tiles, n_e), 0)
    in_e = jnp.logical_and(tio >= tb, tio < tb + ntl).astype(jnp.int32)
    eio = jax.lax.broadcasted_iota(jnp.int32, (n_tiles, n_e), 1)
    te_ref[...] = jnp.sum(in_e * eio, axis=1, keepdims=True)
    act_ref[...] = jnp.sum(in_e, axis=1, keepdims=True)


def _grouped_kernel(te_ref, act_ref, xs_ref, w1_ref, w3_ref, w2_ref,
                    ys_ref, acc_ref):
    i = pl.program_id(0)
    s = pl.program_id(1)

    @pl.when(act_ref[i] == 1)
    def _():
        xb = xs_ref[...]
        a1 = jax.lax.dot_general(
            xb, w1_ref[0], (((1,), (1,)), ((), ())),
            preferred_element_type=jnp.float32)
        a3 = jax.lax.dot_general(
            xb, w3_ref[0], (((1,), (1,)), ((), ())),
            preferred_element_type=jnp.float32)
        ch = (a1 * jax.nn.sigmoid(a1)) * a3

        @pl.when(s == 0)
        def _():
            acc_ref[...] = ch

        @pl.when(s == 1)
        def _():
            av = jnp.concatenate([acc_ref[...], ch], axis=1)
            ys_ref[...] = jax.lax.dot_general(
                av, w2_ref[0], (((1,), (1,)), ((), ())),
                preferred_element_type=jnp.float32)


def _shared_kernel(x_ref, w1_ref, w3_ref, w2_ref, g0_ref, g1_ref,
                   w0_ref, w1c_ref, out_ref, acc_ref):
    s = pl.program_id(1)

    xb = x_ref[...]
    a1 = jax.lax.dot_general(
        xb, w1_ref[...], (((1,), (1,)), ((), ())),
        preferred_element_type=jnp.float32)
    a3 = jax.lax.dot_general(
        xb, w3_ref[...], (((1,), (1,)), ((), ())),
        preferred_element_type=jnp.float32)
    ch = (a1 * jax.nn.sigmoid(a1)) * a3

    @pl.when(s == 0)
    def _():
        acc_ref[...] = ch

    @pl.when(s == 1)
    def _():
        av = jnp.concatenate([acc_ref[...], ch], axis=1)
        sh = jax.lax.dot_general(
            av, w2_ref[...], (((1,), (1,)), ((), ())),
            preferred_element_type=jnp.float32)
        out_ref[...] = (sh + w0_ref[...] * g0_ref[...]
                        + w1c_ref[...] * g1_ref[...])


def _sc_dispatch(x, d0f, d1f, npad):
    t, d = x.shape
    nc, ns = 2, 16
    nw = nc * ns
    ch = t // nw
    mesh = plsc.VectorSubcoreMesh(core_axis_name="c", subcore_axis_name="s")

    @functools.partial(
        pl.kernel, mesh=mesh,
        out_type=jax.ShapeDtypeStruct((npad, d), jnp.float32),
        scratch_types=[pltpu.VMEM((ch, d), jnp.float32),
                       pltpu.VMEM((ch,), jnp.int32),
                       pltpu.SemaphoreType.DMA])
    def k(x_hbm, d0_hbm, d1_hbm, xs_hbm, xbuf, ibuf, sem):
        wid = lax.axis_index("s") * nc + lax.axis_index("c")
        base = wid * ch
        pltpu.async_copy(x_hbm.at[pl.ds(base, ch)], xbuf, sem).wait()
        pltpu.async_copy(d0_hbm.at[pl.ds(base, ch)], ibuf, sem).wait()
        pltpu.async_copy(xbuf, xs_hbm.at[ibuf], sem).wait()
        pltpu.async_copy(d1_hbm.at[pl.ds(base, ch)], ibuf, sem).wait()
        pltpu.async_copy(xbuf, xs_hbm.at[ibuf], sem).wait()

    return k(x, d0f, d1f)


def _sc_combine(ys, d0f, d1f):
    t = d0f.shape[0]
    d = ys.shape[1]
    nc, ns = 2, 16
    nw = nc * ns
    ch = t // nw
    mesh = plsc.VectorSubcoreMesh(core_axis_name="c", subcore_axis_name="s")

    @functools.partial(
        pl.kernel, mesh=mesh,
        out_type=[jax.ShapeDtypeStruct((t, d), jnp.float32),
                  jax.ShapeDtypeStruct((t, d), jnp.float32)],
        scratch_types=[pltpu.VMEM((ch, d), jnp.float32),
                       pltpu.VMEM((ch,), jnp.int32),
                       pltpu.SemaphoreType.DMA])
    def k(ys_hbm, d0_hbm, d1_hbm, g0_hbm, g1_hbm, buf, ibuf, sem):
        wid = lax.axis_index("s") * nc + lax.axis_index("c")
        base = wid * ch
        pltpu.async_copy(d0_hbm.at[pl.ds(base, ch)], ibuf, sem).wait()
        pltpu.async_copy(ys_hbm.at[ibuf], buf, sem).wait()
        pltpu.async_copy(buf, g0_hbm.at[pl.ds(base, ch)], sem).wait()
        pltpu.async_copy(d1_hbm.at[pl.ds(base, ch)], ibuf, sem).wait()
        pltpu.async_copy(ys_hbm.at[ibuf], buf, sem).wait()
        pltpu.async_copy(buf, g1_hbm.at[pl.ds(base, ch)], sem).wait()

    return k(ys, d0f, d1f)


@jax.jit
def kernel(x, gate_w, w1, w3, w2, ws1, ws3, ws2):
    t, dim = x.shape
    n_exp, hid, _ = w1.shape
    n_k = 2
    n_tiles = (t * n_k) // _R + n_exp
    npad = n_tiles * _R

    d0, d1, w0c, w1c, te, act = pl.pallas_call(
        functools.partial(_gate_route_kernel, r_tile=_R, n_tiles=n_tiles),
        out_shape=[
            jax.ShapeDtypeStruct((t, 1), jnp.int32),
            jax.ShapeDtypeStruct((t, 1), jnp.int32),
            jax.ShapeDtypeStruct((t, 1), jnp.float32),
            jax.ShapeDtypeStruct((t, 1), jnp.float32),
            jax.ShapeDtypeStruct((n_tiles, 1), jnp.int32),
            jax.ShapeDtypeStruct((n_tiles, 1), jnp.int32),
        ],
    )(x, gate_w)

    d0f = d0.reshape(t)
    d1f = d1.reshape(t)
    te_f = te.reshape(n_tiles)
    act_f = act.reshape(n_tiles)

    xs = _sc_dispatch(x, d0f, d1f, npad)

    hid_p = hid

    tt = max(1, t // 256)
    t_blk = t // tt
    h_blk = hid_p // 2

    grid_spec = pltpu.PrefetchScalarGridSpec(
        num_scalar_prefetch=2,
        grid=(n_tiles, 2),
        in_specs=[
            pl.BlockSpec((_R, dim), lambda i, s, te_r, act_r: (i, 0)),
            pl.BlockSpec((1, h_blk, dim),
                         lambda i, s, te_r, act_r: (te_r[i], s, 0)),
            pl.BlockSpec((1, h_blk, dim),
                         lambda i, s, te_r, act_r: (te_r[i], s, 0)),
            pl.BlockSpec((1, dim, hid_p),
                         lambda i, s, te_r, act_r: (te_r[i], 0, 0)),
        ],
        out_specs=pl.BlockSpec((_R, dim), lambda i, s, te_r, act_r: (i, 0)),
        scratch_shapes=[pltpu.VMEM((_R, h_blk), jnp.float32)],
    )
    ys = pl.pallas_call(
        _grouped_kernel,
        grid_spec=grid_spec,
        out_shape=jax.ShapeDtypeStruct((npad, dim), jnp.float32),
    )(te_f, act_f, xs, w1, w3, w2)

    g0, g1 = _sc_combine(ys, d0f, d1f)

    out = pl.pallas_call(
        _shared_kernel,
        grid=(tt, 2),
        in_specs=[
            pl.BlockSpec((t_blk, dim), lambda i, h: (i, 0)),
            pl.BlockSpec((h_blk, dim), lambda i, h: (h, 0)),
            pl.BlockSpec((h_blk, dim), lambda i, h: (h, 0)),
            pl.BlockSpec((dim, hid_p), lambda i, h: (0, 0)),
            pl.BlockSpec((t_blk, dim), lambda i, h: (i, 0)),
            pl.BlockSpec((t_blk, dim), lambda i, h: (i, 0)),
            pl.BlockSpec((t_blk, 1), lambda i, h: (i, 0)),
            pl.BlockSpec((t_blk, 1), lambda i, h: (i, 0)),
        ],
        out_specs=pl.BlockSpec((t_blk, dim), lambda i, h: (i, 0)),
        out_shape=jax.ShapeDtypeStruct((t, dim), jnp.float32),
        scratch_shapes=[pltpu.VMEM((t_blk, h_blk), jnp.float32)],
    )(x, ws1, ws3, ws2, g0, g1, w0c, w1c)
    return out
